# baseline (speedup 1.0000x reference)
import functools

import jax
import jax.numpy as jnp
from jax import lax
from jax.experimental import pallas as pl
from jax.experimental.pallas import tpu as pltpu
from jax.experimental.pallas import tpu_sc as plsc

N_NODES = 10000
N_EDGES = 320000
D_FEAT = 128
D_EDGE = 16

NC, NS = 2, 16
NSEG = 10240
NB = N_EDGES // 128
BPC = NB // NC
JCH = 125
NLOAD = BPC // JCH
GP = 128 // 16


def _sc_body(ea4, ei3, acc_out, cnt_out, vbufs, dbufs, acc, cnt,
             sem_v, sem_d):
    c = lax.axis_index("c")
    s = lax.axis_index("s")
    g = s // 8
    ss = s % 8

    zero16 = jnp.zeros((16,), jnp.float32)
    one16 = jnp.ones((16,), jnp.float32)

    @pl.loop(0, NSEG // 16)
    def _(i):
        acc[pl.ds(i * 16, 16)] = zero16
        cnt[pl.ds(i * 16, 16)] = zero16

    def start(l):
        base = c * BPC + l * JCH
        cv = pltpu.async_copy(ea4.at[g, pl.ds(base, JCH), ss],
                              vbufs.at[l % 2], sem_v)
        cd = pltpu.async_copy(ei3.at[pl.ds(base, JCH), 1],
                              dbufs.at[l % 2], sem_d)
        return cv, cd

    pend = start(0)
    for l in range(NLOAD):
        pend[0].wait()
        pend[1].wait()
        if l + 1 < NLOAD:
            pend = start(l + 1)
        @plsc.parallel_loop(0, JCH, step=1, unroll=8)
        def _(r):
            for k in range(GP):
                idx = dbufs[l % 2, r, pl.ds(k * 16, 16)]
                val = vbufs[l % 2, r, pl.ds(k * 16, 16)]
                plsc.addupdate_scatter(acc, [idx], val)

        @plsc.parallel_loop(0, 8, step=1, unroll=2)
        def _(i):
            row = 8 * s + i

            @pl.when(row < JCH)
            def _():
                for k in range(GP):
                    idx = dbufs[l % 2, row, pl.ds(k * 16, 16)]
                    plsc.addupdate_scatter(cnt, [idx], one16)

    pltpu.sync_copy(acc, acc_out.at[c, s])
    pltpu.sync_copy(cnt, cnt_out.at[c, s])


_sc_scatter = functools.partial(
    pl.kernel,
    out_type=(
        jax.ShapeDtypeStruct((NC, NS, NSEG), jnp.float32),
        jax.ShapeDtypeStruct((NC, NS, NSEG), jnp.float32),
    ),
    mesh=plsc.VectorSubcoreMesh(core_axis_name="c", subcore_axis_name="s",
                                num_cores=NC, num_subcores=NS),
    compiler_params=pltpu.CompilerParams(use_tc_tiling_on_sc=False,
                                         needs_layout_passes=False),
    scratch_types=(
        pltpu.VMEM((2, JCH, 128), jnp.float32),
        pltpu.VMEM((2, JCH, 128), jnp.int32),
        pltpu.VMEM((NSEG,), jnp.float32),
        pltpu.VMEM((NSEG,), jnp.float32),
        pltpu.SemaphoreType.DMA,
        pltpu.SemaphoreType.DMA,
    ),
)(_sc_body)


ROWS = 2048


def _tc_body(x_ref, acc_ref, cnt_ref, g_ref, W_ref, b_ref, o_ref):
    aT = acc_ref[0] + acc_ref[1]
    ct = cnt_ref[0] + cnt_ref[1]
    cnt = jnp.sum(ct, axis=0, keepdims=True)
    aggT = aT / jnp.maximum(cnt, 1.0)
    const = (jnp.dot(g_ref[...], W_ref[D_EDGE + D_FEAT:, :],
                     preferred_element_type=jnp.float32) + b_ref[...])
    agg_w = lax.dot_general(aggT, W_ref[0:D_EDGE, :],
                            (((0,), (0,)), ((), ())),
                            preferred_element_type=jnp.float32)
    o_ref[...] = (
        agg_w
        + jnp.dot(x_ref[...], W_ref[D_EDGE:D_EDGE + D_FEAT, :],
                  preferred_element_type=jnp.float32)
        + const)


_tc_finish = pl.pallas_call(
    _tc_body,
    grid=(NSEG // ROWS,),
    in_specs=[
        pl.BlockSpec((ROWS, D_FEAT), lambda i: (i, 0)),
        pl.BlockSpec((NC, NS, ROWS), lambda i: (0, 0, i)),
        pl.BlockSpec((NC, NS, ROWS), lambda i: (0, 0, i)),
        pl.BlockSpec((1, D_FEAT), lambda i: (0, 0)),
        pl.BlockSpec((D_EDGE + D_FEAT + D_FEAT, D_FEAT), lambda i: (0, 0)),
        pl.BlockSpec((1, D_FEAT), lambda i: (0, 0)),
    ],
    out_specs=pl.BlockSpec((ROWS, D_FEAT), lambda i: (i, 0)),
    out_shape=jax.ShapeDtypeStruct((N_NODES, D_FEAT), jnp.float32),
)


def kernel(x, edge_attr, edge_index, g, W, b):
    ea4 = edge_attr.reshape(NB, 128, 2, 8).transpose(2, 0, 3, 1)
    ei3 = edge_index.reshape(2, NB, 128).transpose(1, 0, 2)
    acc, cnt = _sc_scatter(ea4, ei3)
    return _tc_finish(x, acc, cnt, g.reshape(1, D_FEAT),
                      W, b.reshape(1, D_FEAT))


# speedup vs baseline: 1.0357x; 1.0357x over previous
import functools

import jax
import jax.numpy as jnp
from jax import lax
from jax.experimental import pallas as pl
from jax.experimental.pallas import tpu as pltpu
from jax.experimental.pallas import tpu_sc as plsc

N_NODES = 10000
N_EDGES = 320000
D_FEAT = 128
D_EDGE = 16

NC, NS = 2, 16
NSEG = 10240
NB = N_EDGES // 128
BPC = NB // NC
JCH = 125
NLOAD = BPC // JCH
GP = 128 // 16


def _sc_body(ea4, ei3, acc_out, cnt_out, vbufs, dbufs, acc, cnt,
             sem_v, sem_d):
    c = lax.axis_index("c")
    s = lax.axis_index("s")
    g = s // 8
    ss = s % 8

    zero16 = jnp.zeros((16,), jnp.float32)
    one16 = jnp.ones((16,), jnp.float32)

    @pl.loop(0, NSEG // 16)
    def _(i):
        acc[pl.ds(i * 16, 16)] = zero16
        cnt[pl.ds(i * 16, 16)] = zero16

    def start(l):
        base = c * BPC + l * JCH
        cv = pltpu.async_copy(ea4.at[g, pl.ds(base, JCH), ss],
                              vbufs.at[l % 2], sem_v)
        cd = pltpu.async_copy(ei3.at[pl.ds(base, JCH), 1],
                              dbufs.at[l % 2], sem_d)
        return cv, cd

    pend = start(0)
    for l in range(NLOAD):
        pend[0].wait()
        pend[1].wait()
        if l + 1 < NLOAD:
            pend = start(l + 1)
        @plsc.parallel_loop(0, JCH, step=1, unroll=5)
        def _(r):
            for k in range(GP):
                idx = dbufs[l % 2, r, pl.ds(k * 16, 16)]
                val = vbufs[l % 2, r, pl.ds(k * 16, 16)]
                plsc.addupdate_scatter(acc, [idx], val)

        @plsc.parallel_loop(0, 8, step=1, unroll=2)
        def _(i):
            row = 8 * s + i

            @pl.when(row < JCH)
            def _():
                for k in range(GP):
                    idx = dbufs[l % 2, row, pl.ds(k * 16, 16)]
                    plsc.addupdate_scatter(cnt, [idx], one16)

    pltpu.sync_copy(acc, acc_out.at[c, s])
    pltpu.sync_copy(cnt, cnt_out.at[c, s])


_sc_scatter = functools.partial(
    pl.kernel,
    out_type=(
        jax.ShapeDtypeStruct((NC, NS, NSEG), jnp.float32),
        jax.ShapeDtypeStruct((NC, NS, NSEG), jnp.float32),
    ),
    mesh=plsc.VectorSubcoreMesh(core_axis_name="c", subcore_axis_name="s",
                                num_cores=NC, num_subcores=NS),
    compiler_params=pltpu.CompilerParams(use_tc_tiling_on_sc=False,
                                         needs_layout_passes=False),
    scratch_types=(
        pltpu.VMEM((2, JCH, 128), jnp.float32),
        pltpu.VMEM((2, JCH, 128), jnp.int32),
        pltpu.VMEM((NSEG,), jnp.float32),
        pltpu.VMEM((NSEG,), jnp.float32),
        pltpu.SemaphoreType.DMA,
        pltpu.SemaphoreType.DMA,
    ),
)(_sc_body)


ROWS = 2048


def _tc_body(x_ref, acc_ref, cnt_ref, g_ref, W_ref, b_ref, o_ref):
    aT = acc_ref[0] + acc_ref[1]
    ct = cnt_ref[0] + cnt_ref[1]
    cnt = jnp.sum(ct, axis=0, keepdims=True)
    aggT = aT / jnp.maximum(cnt, 1.0)
    const = (jnp.dot(g_ref[...], W_ref[D_EDGE + D_FEAT:, :],
                     preferred_element_type=jnp.float32) + b_ref[...])
    agg_w = lax.dot_general(aggT, W_ref[0:D_EDGE, :],
                            (((0,), (0,)), ((), ())),
                            preferred_element_type=jnp.float32)
    o_ref[...] = (
        agg_w
        + jnp.dot(x_ref[...], W_ref[D_EDGE:D_EDGE + D_FEAT, :],
                  preferred_element_type=jnp.float32)
        + const)


_tc_finish = pl.pallas_call(
    _tc_body,
    grid=(NSEG // ROWS,),
    in_specs=[
        pl.BlockSpec((ROWS, D_FEAT), lambda i: (i, 0)),
        pl.BlockSpec((NC, NS, ROWS), lambda i: (0, 0, i)),
        pl.BlockSpec((NC, NS, ROWS), lambda i: (0, 0, i)),
        pl.BlockSpec((1, D_FEAT), lambda i: (0, 0)),
        pl.BlockSpec((D_EDGE + D_FEAT + D_FEAT, D_FEAT), lambda i: (0, 0)),
        pl.BlockSpec((1, D_FEAT), lambda i: (0, 0)),
    ],
    out_specs=pl.BlockSpec((ROWS, D_FEAT), lambda i: (i, 0)),
    out_shape=jax.ShapeDtypeStruct((N_NODES, D_FEAT), jnp.float32),
)


def kernel(x, edge_attr, edge_index, g, W, b):
    ea4 = edge_attr.reshape(NB, 128, 2, 8).transpose(2, 0, 3, 1)
    ei3 = edge_index.reshape(2, NB, 128).transpose(1, 0, 2)
    acc, cnt = _sc_scatter(ea4, ei3)
    return _tc_finish(x, acc, cnt, g.reshape(1, D_FEAT),
                      W, b.reshape(1, D_FEAT))
